# baseline (speedup 1.0000x reference)
import functools

import jax
import jax.numpy as jnp
from jax import lax
from jax.experimental import pallas as pl
from jax.experimental.pallas import tpu as pltpu
from jax.experimental.pallas import tpu_sc as plsc

N = 10000
NP = 10112
H = 128
E = 320000
EPAD = 327680
PADE = EPAD - E
CH1 = EPAD // (32 * 128)
NSC1 = CH1 // 8
RPS = NP // 16


def _deg_body(dst_hbm, ones_hbm, zero_hbm, out0_hbm, out1_hbm, idx_v, ones_v,
              acc_sh):
    c = lax.axis_index("c")
    s = lax.axis_index("s")
    pltpu.sync_copy(dst_hbm.at[c, s], idx_v)
    pltpu.sync_copy(ones_hbm, ones_v)

    @pl.when(s == 0)
    def _():
        pltpu.sync_copy(zero_hbm, acc_sh)

    plsc.subcore_barrier()

    def body(j, carry):
        pltpu.sync_copy(ones_v, acc_sh.at[idx_v.at[j]], add=True)
        return carry

    lax.fori_loop(0, CH1, body, 0)
    plsc.subcore_barrier()

    @pl.when((s == 0) & (c == 0))
    def _():
        pltpu.sync_copy(acc_sh, out0_hbm)

    @pl.when((s == 0) & (c == 1))
    def _():
        pltpu.sync_copy(acc_sh, out1_hbm)


@functools.cache
def _deg_call():
    return pl.kernel(
        _deg_body,
        out_type=(jax.ShapeDtypeStruct((NP,), jnp.float32),
                  jax.ShapeDtypeStruct((NP,), jnp.float32)),
        mesh=plsc.VectorSubcoreMesh(core_axis_name="c", subcore_axis_name="s"),
        scratch_types=[
            pltpu.VMEM((CH1, 128), jnp.int32),
            pltpu.VMEM((128,), jnp.float32),
            pltpu.VMEM_SHARED((NP,), jnp.float32),
        ],
    )


def _gcn_body(y_hbm, src_hbm, dst_hbm, zero_hbm, out_hbm,
              si0, si1, di0, di1, buf0, buf1, acc_sh, semg0, semg1, sems0,
              sems1, semi):
    c = lax.axis_index("c")
    s = lax.axis_index("s")
    pltpu.sync_copy(zero_hbm.at[pl.ds(s * RPS, RPS)],
                    acc_sh.at[pl.ds(s * RPS, RPS)])
    plsc.subcore_barrier()
    pltpu.sync_copy(src_hbm.at[c, s, pl.ds(0, 8)], si0)
    pltpu.sync_copy(dst_hbm.at[c, s, pl.ds(0, 8)], di0)

    def process(u, si, di, si_n, di_n):
        @pl.when(u + 1 < NSC1)
        def _():
            pltpu.async_copy(src_hbm.at[c, s, pl.ds((u + 1) * 8, 8)], si_n,
                             semi)
            pltpu.async_copy(dst_hbm.at[c, s, pl.ds((u + 1) * 8, 8)], di_n,
                             semi)

        pltpu.async_copy(y_hbm.at[si.at[0]], buf0, semg0)
        for k in range(8):
            bufk, semgk, semsk = ((buf0, semg0, sems0) if k % 2 == 0
                                  else (buf1, semg1, sems1))
            bufn, semgn, semsn = ((buf1, semg1, sems1) if k % 2 == 0
                                  else (buf0, semg0, sems0))
            pltpu.make_async_copy(y_hbm.at[si.at[k]], bufk, semgk).wait()
            pltpu.async_copy(bufk, acc_sh.at[di.at[k]], semsk, add=True)
            if k < 7:
                if k > 0:
                    pltpu.make_async_copy(
                        bufn, acc_sh.at[di.at[k]], semsn).wait()
                pltpu.async_copy(y_hbm.at[si.at[k + 1]], bufn, semgn)
        pltpu.make_async_copy(buf0, acc_sh.at[di.at[0]], sems0).wait()
        pltpu.make_async_copy(buf1, acc_sh.at[di.at[0]], sems1).wait()

        @pl.when(u + 1 < NSC1)
        def _():
            pltpu.make_async_copy(src_hbm.at[c, s, pl.ds((u + 1) * 8, 8)],
                                  si_n, semi).wait()
            pltpu.make_async_copy(dst_hbm.at[c, s, pl.ds((u + 1) * 8, 8)],
                                  di_n, semi).wait()

    def body(v, carry):
        process(2 * v, si0, di0, si1, di1)
        process(2 * v + 1, si1, di1, si0, di0)
        return carry

    lax.fori_loop(0, NSC1 // 2, body, 0)
    plsc.subcore_barrier()
    pltpu.sync_copy(acc_sh.at[pl.ds(s * RPS, RPS)],
                    out_hbm.at[c, pl.ds(s * RPS, RPS)])


@functools.cache
def _gcn_call():
    return pl.kernel(
        _gcn_body,
        out_type=jax.ShapeDtypeStruct((2, NP, 128), jnp.float32),
        mesh=plsc.VectorSubcoreMesh(core_axis_name="c", subcore_axis_name="s"),
        scratch_types=[
            pltpu.VMEM((8, 128), jnp.int32),
            pltpu.VMEM((8, 128), jnp.int32),
            pltpu.VMEM((8, 128), jnp.int32),
            pltpu.VMEM((8, 128), jnp.int32),
            pltpu.VMEM((128, 128), jnp.float32),
            pltpu.VMEM((128, 128), jnp.float32),
            pltpu.VMEM_SHARED((NP, 128), jnp.float32),
            pltpu.SemaphoreType.DMA,
            pltpu.SemaphoreType.DMA,
            pltpu.SemaphoreType.DMA,
            pltpu.SemaphoreType.DMA,
            pltpu.SemaphoreType.DMA,
        ],
    )


def _prep_body(x_ref, q_ref, w1g_ref, w1q_ref, deg0_ref, deg1_ref, yg_ref,
               yq_ref, dinv_ref):
    deg = deg0_ref[...] + deg1_ref[...] + 1.0
    dinv = lax.rsqrt(deg)[:, None]
    dinv_ref[...] = dinv
    xw = jnp.dot(x_ref[...], w1g_ref[...], preferred_element_type=jnp.float32)
    qw = jnp.dot(q_ref[...], w1q_ref[...], preferred_element_type=jnp.float32)
    yg_ref[...] = xw * dinv
    yq_ref[...] = qw * dinv


def _prep_call(xp, qp, w1g, w1q, deg0, deg1):
    return pl.pallas_call(
        _prep_body,
        out_shape=(
            jax.ShapeDtypeStruct((NP, 128), jnp.float32),
            jax.ShapeDtypeStruct((NP, 128), jnp.float32),
            jax.ShapeDtypeStruct((NP, 1), jnp.float32),
        ),
    )(xp, qp, w1g, w1q, deg0, deg1)


def _bn_relu_rows(t, gam, bet):
    tv = t[:N]
    m = jnp.mean(tv, axis=0, keepdims=True)
    v = jnp.mean((tv - m) ** 2, axis=0, keepdims=True)
    return jnp.maximum((tv - m) * lax.rsqrt(v + 1e-5) * gam + bet, 0.0)


def _mid1_body(s_ref, y_ref, dinv_ref, b_ref, gam_ref, bet_ref, xh_ref):
    t = dinv_ref[...] * (s_ref[0] + s_ref[1] + y_ref[...]) + b_ref[...]
    xh_ref[...] = _bn_relu_rows(t, gam_ref[...], bet_ref[...])


def _mid1_call(*args):
    return pl.pallas_call(
        _mid1_body,
        out_shape=jax.ShapeDtypeStruct((N, 128), jnp.float32),
    )(*args)


def _gru_body(xh_ref, dinv_ref, h0_ref, wihT_ref, whhT_ref, bi_ref, bhn_ref,
              w2_ref, y2_ref, hT_ref, gi_ref):
    gi_ref[...] = jnp.dot(xh_ref[...], wihT_ref[...],
                          preferred_element_type=jnp.float32) + bi_ref[...]
    whhT = whhT_ref[...]
    bhn = bhn_ref[...]
    y2_ref[pl.ds(N, NP - N), :] = jnp.zeros((NP - N, 128), jnp.float32)

    def step(t, h):
        gh = jnp.dot(h, whhT, preferred_element_type=jnp.float32)
        gi_t = gi_ref[pl.ds(t, 1), :]
        thr = jnp.tanh((gi_t[:, 0:128] + gh[:, 0:128]) * 0.5)
        thz = jnp.tanh((gi_t[:, 128:256] + gh[:, 128:256]) * 0.5)
        hb = 0.5 * (gh[:, 256:384] + bhn)
        a = gi_t[:, 256:384] + hb
        n = jnp.tanh(a + thr * hb)
        t1 = (0.5 * h) + (0.5 * h) * thz
        t2 = 0.5 - 0.5 * thz
        h2 = t1 + t2 * n
        y2_ref[pl.ds(t, 1), :] = h2
        return h2

    def step8(v, h):
        for k in range(8):
            h = step(8 * v + k, h)
        return h

    hT = lax.fori_loop(0, N // 8, step8, h0_ref[...])
    hT_ref[...] = hT
    y2_ref[...] = jnp.dot(y2_ref[...], w2_ref[...],
                          preferred_element_type=jnp.float32) * dinv_ref[...]


def _gru_call(*args):
    return pl.pallas_call(
        _gru_body,
        out_shape=(
            jax.ShapeDtypeStruct((NP, 128), jnp.float32),
            jax.ShapeDtypeStruct((1, 128), jnp.float32),
        ),
        scratch_shapes=[pltpu.VMEM((N, 384), jnp.float32)],
    )(*args)


def _post_body(s2g_ref, s2q_ref, y2g_ref, y2q_ref, dinv_ref, bg_ref, bq_ref,
               g2g_ref, g2b_ref, q2g_ref, q2b_ref, wlin_ref, blin_ref,
               out_ref):
    dinv = dinv_ref[...]
    og = _bn_relu_rows(dinv * (s2g_ref[0] + s2g_ref[1] + y2g_ref[...])
                       + bg_ref[...], g2g_ref[...], g2b_ref[...])
    oq = _bn_relu_rows(dinv * (s2q_ref[0] + s2q_ref[1] + y2q_ref[...])
                       + bq_ref[...], q2g_ref[...], q2b_ref[...])
    res = jnp.dot(og + oq, wlin_ref[...],
                  preferred_element_type=jnp.float32) + blin_ref[...]
    out_ref[...] = jax.nn.sigmoid(res)


def _post_call(*args):
    return pl.pallas_call(
        _post_body,
        out_shape=jax.ShapeDtypeStruct((N, 1), jnp.float32),
    )(*args)


def kernel(x, query, edge_index, W1g, b1g, W1q, b1q, g1_gamma, g1_beta,
           q1_gamma, q1_beta, Wih_g, Whh_g, bih_g, bhh_g, Wih_q, Whh_q,
           bih_q, bhh_q, w_att, W2g, b2g, W2q, b2q, g2_gamma, g2_beta,
           q2_gamma, q2_beta, Wlin, blin):
    src = edge_index[0]
    dst = edge_index[1]
    pad_idx = (jnp.arange(PADE, dtype=jnp.int32) % 16) + N
    src2 = jnp.concatenate([src, pad_idx]).reshape(2, 16, CH1, 128)
    dst2 = jnp.concatenate([dst, pad_idx]).reshape(2, 16, CH1, 128)
    xp = jnp.pad(x, ((0, NP - N), (0, 0)))
    qp = jnp.pad(query, ((0, NP - N), (0, 0)))
    ones128 = jnp.ones((128,), jnp.float32)
    zNP = jnp.zeros((NP,), jnp.float32)
    zNP128 = jnp.zeros((NP, 128), jnp.float32)

    deg0, deg1 = _deg_call()(dst2, ones128, zNP)
    y1g, y1q, dinv = _prep_call(xp, qp, W1g, W1q, deg0, deg1)
    s1g = _gcn_call()(y1g, src2, dst2, zNP128)
    s1q = _gcn_call()(y1q, src2, dst2, zNP128)

    r2 = lambda v: v.reshape(1, -1)
    bi_g = r2(jnp.concatenate([bih_g[:2 * H] + bhh_g[:2 * H], bih_g[2 * H:]]))
    bi_q = r2(jnp.concatenate([bih_q[:2 * H] + bhh_q[:2 * H], bih_q[2 * H:]]))
    h0 = jnp.zeros((1, 128), jnp.float32)

    xh_g = _mid1_call(s1g, y1g, dinv, r2(b1g), r2(g1_gamma), r2(g1_beta))
    y2g, hg = _gru_call(xh_g, dinv, h0, Wih_g.T, Whh_g.T, bi_g,
                        r2(bhh_g[2 * H:]), W2g)
    s2g = _gcn_call()(y2g, src2, dst2, zNP128)
    xh_q = _mid1_call(s1q, y1q, dinv, r2(b1q), r2(q1_gamma), r2(q1_beta))
    y2q, hq = _gru_call(xh_q, dinv, hg, Wih_q.T, Whh_q.T, bi_q,
                        r2(bhh_q[2 * H:]), W2q)
    s2q = _gcn_call()(y2q, src2, dst2, zNP128)
    out = _post_call(s2g, s2q, y2g, y2q, dinv, r2(b2g), r2(b2q), r2(g2_gamma),
                     r2(g2_beta), r2(q2_gamma), r2(q2_beta), Wlin, r2(blin))
    return out[:, 0], hg, hq


# speedup vs baseline: 1.0066x; 1.0066x over previous
import functools

import jax
import jax.numpy as jnp
from jax import lax
from jax.experimental import pallas as pl
from jax.experimental.pallas import tpu as pltpu
from jax.experimental.pallas import tpu_sc as plsc

N = 10000
NP = 10112
H = 128
E = 320000
EPAD = 327680
PADE = EPAD - E
CH1 = EPAD // (32 * 128)
NSC1 = CH1 // 8
RPS = NP // 16


def _deg_body(dst_hbm, ones_hbm, zero_hbm, out0_hbm, out1_hbm, idx_v, ones_v,
              acc_sh):
    c = lax.axis_index("c")
    s = lax.axis_index("s")
    pltpu.sync_copy(dst_hbm.at[c, s], idx_v)
    pltpu.sync_copy(ones_hbm, ones_v)

    @pl.when(s == 0)
    def _():
        pltpu.sync_copy(zero_hbm, acc_sh)

    plsc.subcore_barrier()

    def body(j, carry):
        pltpu.sync_copy(ones_v, acc_sh.at[idx_v.at[j]], add=True)
        return carry

    lax.fori_loop(0, CH1, body, 0)
    plsc.subcore_barrier()

    @pl.when((s == 0) & (c == 0))
    def _():
        pltpu.sync_copy(acc_sh, out0_hbm)

    @pl.when((s == 0) & (c == 1))
    def _():
        pltpu.sync_copy(acc_sh, out1_hbm)


@functools.cache
def _deg_call():
    return pl.kernel(
        _deg_body,
        out_type=(jax.ShapeDtypeStruct((NP,), jnp.float32),
                  jax.ShapeDtypeStruct((NP,), jnp.float32)),
        mesh=plsc.VectorSubcoreMesh(core_axis_name="c", subcore_axis_name="s"),
        scratch_types=[
            pltpu.VMEM((CH1, 128), jnp.int32),
            pltpu.VMEM((128,), jnp.float32),
            pltpu.VMEM_SHARED((NP,), jnp.float32),
        ],
    )


def _gcn_body(y_hbm, src_hbm, dst_hbm, zero_hbm, out_hbm,
              si0, si1, di0, di1, buf0, buf1, acc_sh, semg0, semg1, sems0,
              sems1, semi):
    c = lax.axis_index("c")
    s = lax.axis_index("s")
    pltpu.sync_copy(zero_hbm.at[pl.ds(s * RPS, RPS)],
                    acc_sh.at[pl.ds(s * RPS, RPS)])
    plsc.subcore_barrier()
    pltpu.sync_copy(src_hbm.at[c, s, pl.ds(0, 8)], si0)
    pltpu.sync_copy(dst_hbm.at[c, s, pl.ds(0, 8)], di0)

    def process(u, si, di, si_n, di_n):
        @pl.when(u + 1 < NSC1)
        def _():
            pltpu.async_copy(src_hbm.at[c, s, pl.ds((u + 1) * 8, 8)], si_n,
                             semi)
            pltpu.async_copy(dst_hbm.at[c, s, pl.ds((u + 1) * 8, 8)], di_n,
                             semi)

        pltpu.async_copy(y_hbm.at[si.at[0]], buf0, semg0)
        for k in range(8):
            bufk, semgk, semsk = ((buf0, semg0, sems0) if k % 2 == 0
                                  else (buf1, semg1, sems1))
            bufn, semgn, semsn = ((buf1, semg1, sems1) if k % 2 == 0
                                  else (buf0, semg0, sems0))
            pltpu.make_async_copy(y_hbm.at[si.at[k]], bufk, semgk).wait()
            pltpu.async_copy(bufk, acc_sh.at[di.at[k]], semsk, add=True)
            if k < 7:
                if k > 0:
                    pltpu.make_async_copy(
                        bufn, acc_sh.at[di.at[k]], semsn).wait()
                pltpu.async_copy(y_hbm.at[si.at[k + 1]], bufn, semgn)
        pltpu.make_async_copy(buf0, acc_sh.at[di.at[0]], sems0).wait()
        pltpu.make_async_copy(buf1, acc_sh.at[di.at[0]], sems1).wait()

        @pl.when(u + 1 < NSC1)
        def _():
            pltpu.make_async_copy(src_hbm.at[c, s, pl.ds((u + 1) * 8, 8)],
                                  si_n, semi).wait()
            pltpu.make_async_copy(dst_hbm.at[c, s, pl.ds((u + 1) * 8, 8)],
                                  di_n, semi).wait()

    def body(v, carry):
        process(2 * v, si0, di0, si1, di1)
        process(2 * v + 1, si1, di1, si0, di0)
        return carry

    lax.fori_loop(0, NSC1 // 2, body, 0)
    plsc.subcore_barrier()
    pltpu.sync_copy(acc_sh.at[pl.ds(s * RPS, RPS)],
                    out_hbm.at[c, pl.ds(s * RPS, RPS)])


@functools.cache
def _gcn_call():
    return pl.kernel(
        _gcn_body,
        out_type=jax.ShapeDtypeStruct((2, NP, 128), jnp.float32),
        mesh=plsc.VectorSubcoreMesh(core_axis_name="c", subcore_axis_name="s"),
        scratch_types=[
            pltpu.VMEM((8, 128), jnp.int32),
            pltpu.VMEM((8, 128), jnp.int32),
            pltpu.VMEM((8, 128), jnp.int32),
            pltpu.VMEM((8, 128), jnp.int32),
            pltpu.VMEM((128, 128), jnp.float32),
            pltpu.VMEM((128, 128), jnp.float32),
            pltpu.VMEM_SHARED((NP, 128), jnp.float32),
            pltpu.SemaphoreType.DMA,
            pltpu.SemaphoreType.DMA,
            pltpu.SemaphoreType.DMA,
            pltpu.SemaphoreType.DMA,
            pltpu.SemaphoreType.DMA,
        ],
    )


def _prep_body(x_ref, q_ref, w1g_ref, w1q_ref, deg0_ref, deg1_ref, yg_ref,
               yq_ref, dinv_ref):
    deg = deg0_ref[...] + deg1_ref[...] + 1.0
    dinv = lax.rsqrt(deg)[:, None]
    dinv_ref[...] = dinv
    xw = jnp.dot(x_ref[...], w1g_ref[...], preferred_element_type=jnp.float32)
    qw = jnp.dot(q_ref[...], w1q_ref[...], preferred_element_type=jnp.float32)
    yg_ref[...] = xw * dinv
    yq_ref[...] = qw * dinv


def _prep_call(xp, qp, w1g, w1q, deg0, deg1):
    return pl.pallas_call(
        _prep_body,
        out_shape=(
            jax.ShapeDtypeStruct((NP, 128), jnp.float32),
            jax.ShapeDtypeStruct((NP, 128), jnp.float32),
            jax.ShapeDtypeStruct((NP, 1), jnp.float32),
        ),
    )(xp, qp, w1g, w1q, deg0, deg1)


def _bn_relu_rows(t, gam, bet):
    tv = t[:N]
    m = jnp.mean(tv, axis=0, keepdims=True)
    v = jnp.mean((tv - m) ** 2, axis=0, keepdims=True)
    return jnp.maximum((tv - m) * lax.rsqrt(v + 1e-5) * gam + bet, 0.0)


def _mid1_body(s_ref, y_ref, dinv_ref, b_ref, gam_ref, bet_ref, xh_ref):
    t = dinv_ref[...] * (s_ref[0] + s_ref[1] + y_ref[...]) + b_ref[...]
    xh_ref[...] = _bn_relu_rows(t, gam_ref[...], bet_ref[...])


def _mid1_call(*args):
    return pl.pallas_call(
        _mid1_body,
        out_shape=jax.ShapeDtypeStruct((N, 128), jnp.float32),
    )(*args)


def _gru_body(xh_ref, dinv_ref, h0_ref, wihT_ref, whhT_ref, bi_ref, bhn_ref,
              w2_ref, y2_ref, hT_ref, gi_ref):
    gi_ref[...] = jnp.dot(xh_ref[...], wihT_ref[...],
                          preferred_element_type=jnp.float32) + bi_ref[...]
    whhT = whhT_ref[...]
    bhn = bhn_ref[...]
    y2_ref[pl.ds(N, NP - N), :] = jnp.zeros((NP - N, 128), jnp.float32)

    def step(t, h):
        gh = jnp.dot(h, whhT, preferred_element_type=jnp.float32)
        gi_t = gi_ref[pl.ds(t, 1), :]
        thr = jnp.tanh((gi_t[:, 0:128] + gh[:, 0:128]) * 0.5)
        thz = jnp.tanh((gi_t[:, 128:256] + gh[:, 128:256]) * 0.5)
        hb = 0.5 * (gh[:, 256:384] + bhn)
        a = gi_t[:, 256:384] + hb
        n = jnp.tanh(a + thr * hb)
        t1 = (0.5 * h) + (0.5 * h) * thz
        t2 = 0.5 - 0.5 * thz
        h2 = t1 + t2 * n
        y2_ref[pl.ds(t, 1), :] = h2
        return h2

    def step16(v, h):
        for k in range(16):
            h = step(16 * v + k, h)
        return h

    hT = lax.fori_loop(0, N // 16, step16, h0_ref[...])
    hT_ref[...] = hT
    y2_ref[...] = jnp.dot(y2_ref[...], w2_ref[...],
                          preferred_element_type=jnp.float32) * dinv_ref[...]


def _gru_call(*args):
    return pl.pallas_call(
        _gru_body,
        out_shape=(
            jax.ShapeDtypeStruct((NP, 128), jnp.float32),
            jax.ShapeDtypeStruct((1, 128), jnp.float32),
        ),
        scratch_shapes=[pltpu.VMEM((N, 384), jnp.float32)],
    )(*args)


def _post_body(s2g_ref, s2q_ref, y2g_ref, y2q_ref, dinv_ref, bg_ref, bq_ref,
               g2g_ref, g2b_ref, q2g_ref, q2b_ref, wlin_ref, blin_ref,
               out_ref):
    dinv = dinv_ref[...]
    og = _bn_relu_rows(dinv * (s2g_ref[0] + s2g_ref[1] + y2g_ref[...])
                       + bg_ref[...], g2g_ref[...], g2b_ref[...])
    oq = _bn_relu_rows(dinv * (s2q_ref[0] + s2q_ref[1] + y2q_ref[...])
                       + bq_ref[...], q2g_ref[...], q2b_ref[...])
    res = jnp.dot(og + oq, wlin_ref[...],
                  preferred_element_type=jnp.float32) + blin_ref[...]
    out_ref[...] = jax.nn.sigmoid(res)


def _post_call(*args):
    return pl.pallas_call(
        _post_body,
        out_shape=jax.ShapeDtypeStruct((N, 1), jnp.float32),
    )(*args)


def kernel(x, query, edge_index, W1g, b1g, W1q, b1q, g1_gamma, g1_beta,
           q1_gamma, q1_beta, Wih_g, Whh_g, bih_g, bhh_g, Wih_q, Whh_q,
           bih_q, bhh_q, w_att, W2g, b2g, W2q, b2q, g2_gamma, g2_beta,
           q2_gamma, q2_beta, Wlin, blin):
    src = edge_index[0]
    dst = edge_index[1]
    pad_idx = (jnp.arange(PADE, dtype=jnp.int32) % 16) + N
    src2 = jnp.concatenate([src, pad_idx]).reshape(2, 16, CH1, 128)
    dst2 = jnp.concatenate([dst, pad_idx]).reshape(2, 16, CH1, 128)
    xp = jnp.pad(x, ((0, NP - N), (0, 0)))
    qp = jnp.pad(query, ((0, NP - N), (0, 0)))
    ones128 = jnp.ones((128,), jnp.float32)
    zNP = jnp.zeros((NP,), jnp.float32)
    zNP128 = jnp.zeros((NP, 128), jnp.float32)

    deg0, deg1 = _deg_call()(dst2, ones128, zNP)
    y1g, y1q, dinv = _prep_call(xp, qp, W1g, W1q, deg0, deg1)
    s1g = _gcn_call()(y1g, src2, dst2, zNP128)
    s1q = _gcn_call()(y1q, src2, dst2, zNP128)

    r2 = lambda v: v.reshape(1, -1)
    bi_g = r2(jnp.concatenate([bih_g[:2 * H] + bhh_g[:2 * H], bih_g[2 * H:]]))
    bi_q = r2(jnp.concatenate([bih_q[:2 * H] + bhh_q[:2 * H], bih_q[2 * H:]]))
    h0 = jnp.zeros((1, 128), jnp.float32)

    xh_g = _mid1_call(s1g, y1g, dinv, r2(b1g), r2(g1_gamma), r2(g1_beta))
    y2g, hg = _gru_call(xh_g, dinv, h0, Wih_g.T, Whh_g.T, bi_g,
                        r2(bhh_g[2 * H:]), W2g)
    s2g = _gcn_call()(y2g, src2, dst2, zNP128)
    xh_q = _mid1_call(s1q, y1q, dinv, r2(b1q), r2(q1_gamma), r2(q1_beta))
    y2q, hq = _gru_call(xh_q, dinv, hg, Wih_q.T, Whh_q.T, bi_q,
                        r2(bhh_q[2 * H:]), W2q)
    s2q = _gcn_call()(y2q, src2, dst2, zNP128)
    out = _post_call(s2g, s2q, y2g, y2q, dinv, r2(b2g), r2(b2q), r2(g2_gamma),
                     r2(g2_beta), r2(q2_gamma), r2(q2_beta), Wlin, r2(blin))
    return out[:, 0], hg, hq
